# baseline (speedup 1.0000x reference)
import functools

import jax
import jax.numpy as jnp
from jax import lax
from jax.experimental import pallas as pl
from jax.experimental.pallas import tpu as pltpu
from jax.experimental.pallas import tpu_sc as plsc

B = 4096
L = 50
D = 64
EPS = 1e-5

NC = 2
NS = 16
NW = NC * NS
BPW = B // NW
R = 128
NCHUNK = (BPW * L) // R


def _make_sc_pool():
    mesh = plsc.VectorSubcoreMesh(core_axis_name="c", subcore_axis_name="s")

    @functools.partial(
        pl.kernel,
        mesh=mesh,
        out_type=jax.ShapeDtypeStruct((B, D), jnp.float32),
        compiler_params=pltpu.CompilerParams(use_tc_tiling_on_sc=False),
        scratch_types=[
            pltpu.VMEM((NCHUNK, R), jnp.int32),
            pltpu.VMEM((NCHUNK, R), jnp.int32),
            pltpu.VMEM((R, D), jnp.float32),
            pltpu.VMEM((R, D), jnp.float32),
            pltpu.VMEM_SHARED((NS * BPW, D), jnp.float32),
            pltpu.SemaphoreType.DMA,
            pltpu.SemaphoreType.DMA,
        ],
    )
    def pool(x_hbm, dst_hbm, zeros_hbm, table_hbm, out_hbm,
             idx_v, dst_v, buf0, buf1, acc, sem0, sem1):
        cid = lax.axis_index("c")
        sid = lax.axis_index("s")
        wid = sid * NC + cid
        base = wid * BPW
        abase = sid * BPW

        pltpu.sync_copy(x_hbm.at[wid], idx_v)
        pltpu.sync_copy(dst_hbm.at[sid], dst_v)
        pltpu.sync_copy(zeros_hbm, acc.at[pl.ds(abase, BPW)])

        pltpu.async_copy(table_hbm.at[idx_v.at[0]], buf0, sem0)
        pltpu.async_copy(table_hbm.at[idx_v.at[1]], buf1, sem1)

        def _step(h, c):
            c0 = 2 * h
            c1 = c0 + 1
            pltpu.make_async_copy(table_hbm.at[idx_v.at[0]], buf0, sem0).wait()
            pltpu.sync_copy(buf0, acc.at[dst_v.at[c0]], add=True)

            @pl.when(c0 + 2 < NCHUNK)
            def _():
                pltpu.async_copy(table_hbm.at[idx_v.at[c0 + 2]], buf0, sem0)

            pltpu.make_async_copy(table_hbm.at[idx_v.at[1]], buf1, sem1).wait()
            pltpu.sync_copy(buf1, acc.at[dst_v.at[c1]], add=True)

            @pl.when(c1 + 2 < NCHUNK)
            def _():
                pltpu.async_copy(table_hbm.at[idx_v.at[c1 + 2]], buf1, sem1)

            return c

        lax.fori_loop(0, NCHUNK // 2, _step, 0)

        pltpu.sync_copy(acc.at[pl.ds(abase, BPW)], out_hbm.at[pl.ds(base, BPW)])

    return pool


_sc_pool = _make_sc_pool()


def _dense_body(s_ref, w_ref, b_ref, bg_ref, bb_ref, lg_ref, lb_ref, o_ref):
    pooled = s_ref[...] * (1.0 / L)
    h = lax.dot_general(
        pooled, w_ref[...], (((1,), (1,)), ((), ())),
        preferred_element_type=jnp.float32,
        precision=lax.Precision.HIGHEST,
    ) + b_ref[...]
    mu = jnp.mean(h, axis=0, keepdims=True)
    var = jnp.mean((h - mu) ** 2, axis=0, keepdims=True)
    hbn = (h - mu) / jnp.sqrt(var + EPS) * bg_ref[...] + bb_ref[...]
    m = jnp.mean(hbn, axis=1, keepdims=True)
    v = jnp.mean((hbn - m) ** 2, axis=1, keepdims=True)
    o_ref[...] = (hbn - m) / jnp.sqrt(v + EPS) * lg_ref[...] + lb_ref[...]


def _dense(sums, W, b, bn_gamma, bn_beta, ln_gamma, ln_beta):
    return pl.pallas_call(
        _dense_body,
        out_shape=jax.ShapeDtypeStruct((B, D), jnp.float32),
    )(sums, W, b.reshape(1, D), bn_gamma.reshape(1, D), bn_beta.reshape(1, D),
      ln_gamma.reshape(1, D), ln_beta.reshape(1, D))


def kernel(x, table, W, b, bn_gamma, bn_beta, ln_gamma, ln_beta):
    x_r = x.astype(jnp.int32).reshape(NW, NCHUNK, R)
    local = jnp.arange(BPW * L, dtype=jnp.int32) // L
    dst = (jnp.arange(NS, dtype=jnp.int32)[:, None] * BPW
           + local[None, :]).reshape(NS, NCHUNK, R)
    zeros = jnp.zeros((BPW, D), jnp.float32)
    sums = _sc_pool(x_r, dst, zeros, table)
    return _dense(sums, W, b, bn_gamma, bn_beta, ln_gamma, ln_beta)
